# baseline (speedup 1.0000x reference)
import functools

import jax
import jax.numpy as jnp
from jax import lax
from jax.experimental import pallas as pl
from jax.experimental.pallas import tpu as pltpu
from jax.experimental.pallas import tpu_sc as plsc

_N_FIELDS = 26
_N_DENSE = 13
_DIM = 32
_EMB_W = _N_FIELDS * _DIM
_OUT_W = _EMB_W + _N_DENSE
_PAD_W = _EMB_W + 16
_CH = 128

_VTILE = 128
_VFULL = 781
_VPAD = 100096
_UNITS_PER_F = _VPAD // _VTILE
_TOT_UNITS = _N_FIELDS * _UNITS_PER_F

_MESH = plsc.VectorSubcoreMesh(core_axis_name="c", subcore_axis_name="s")


def _transpose_tile(inb_ref, outb_ref, lane, n_groups):
    for g in range(n_groups):
        for j in range(8):
            v = 4 * g + j // 2
            dlo = 16 * (j % 2)
            vals = plsc.load_gather(
                inb_ref, [lane + dlo, jnp.full((16,), v, jnp.int32)])
            outb_ref[g, pl.ds(16 * j, 16)] = vals


def _relayout_tables(tables):
    vocab = tables.shape[1]
    tv = jnp.transpose(tables, (0, 2, 1))
    tail = lax.slice(tv, (0, 0, _VFULL * _VTILE), (_N_FIELDS, _DIM, vocab))
    info = plsc.get_sparse_core_info()
    NC, NS = info.num_cores, info.num_subcores
    NW = NC * NS
    n_steps = -(-_TOT_UNITS // NW)

    @functools.partial(
        pl.kernel,
        mesh=_MESH,
        compiler_params=pltpu.CompilerParams(use_tc_tiling_on_sc=True,
                                             needs_layout_passes=False),
        out_type=jax.ShapeDtypeStruct((_TOT_UNITS, _DIM, _VTILE),
                                      jnp.float32),
        scratch_types=[
            pltpu.VMEM((2, _DIM, _VTILE), jnp.float32),
            pltpu.VMEM((2, _DIM, _VTILE), jnp.float32),
            pltpu.VMEM((_DIM, _DIM), jnp.float32),
            pltpu.SemaphoreType.DMA,
            pltpu.SemaphoreType.DMA,
        ],
    )
    def sc_relayout(tv_hbm, tail_hbm, out_hbm, inb, outb, tail_v,
                    isem, osem):
        wid = lax.axis_index("s") * NC + lax.axis_index("c")
        lane = lax.iota(jnp.int32, 16)

        def clamp(u):
            return jnp.where(u % _UNITS_PER_F == _VFULL, u - 1, u)

        def slab(u):
            uc = clamp(u)
            f = uc // _UNITS_PER_F
            vb = (uc % _UNITS_PER_F) * _VTILE
            return tv_hbm.at[f, :, pl.ds(vb, _VTILE)]

        def issue_in(u, slot):
            @pl.when(u < _TOT_UNITS)
            def _():
                pltpu.async_copy(slab(u), inb.at[slot], isem)

        issue_in(wid, 0)
        issue_in(wid + NW, 1)

        def body(i, carry):
            for slot in (0, 1):
                u = wid + (2 * i + slot) * NW

                @pl.when(u < _TOT_UNITS)
                def _():
                    pltpu.make_async_copy(
                        slab(u), inb.at[slot], isem).wait()

                    @pl.when(i > 0)
                    def _():
                        pltpu.make_async_copy(
                            outb.at[slot], out_hbm.at[0], osem).wait()

                    _transpose_tile(inb.at[slot], outb.at[slot], lane, 32)
                    pltpu.async_copy(outb.at[slot], out_hbm.at[clamp(u)],
                                     osem)
                    issue_in(u + 2 * NW, slot)
            return carry

        lax.fori_loop(0, (n_steps + 1) // 2, body, None)
        for slot in (0, 1):
            pltpu.make_async_copy(outb.at[slot], out_hbm.at[0], osem).wait()

        @pl.when(wid < _N_FIELDS)
        def _():
            u = wid * _UNITS_PER_F + _VFULL
            src = tail_hbm.at[wid]
            pltpu.async_copy(src, tail_v, isem)
            pltpu.make_async_copy(src, tail_v, isem).wait()
            _transpose_tile(tail_v, outb.at[0], lane, 8)
            pltpu.async_copy(outb.at[0, pl.ds(0, 8), :],
                             out_hbm.at[u, pl.ds(0, 8), :], osem)
            pltpu.make_async_copy(outb.at[0, pl.ds(0, 8), :],
                                  out_hbm.at[u, pl.ds(0, 8), :], osem).wait()

    return sc_relayout(tv, tail).reshape(_N_FIELDS, _VPAD, _DIM)


def kernel(indices, dense, tables):
    B = indices.shape[1]
    info = plsc.get_sparse_core_info()
    NC, NS = info.num_cores, info.num_subcores
    NW = NC * NS
    rows_per_w = B // NW
    n_chunks = rows_per_w // _CH

    @functools.partial(
        pl.kernel,
        mesh=_MESH,
        compiler_params=pltpu.CompilerParams(use_tc_tiling_on_sc=False),
        out_type=jax.ShapeDtypeStruct((B, _PAD_W), jnp.float32),
        scratch_types=[
            pltpu.VMEM((_N_FIELDS, _CH), jnp.int32),
            pltpu.VMEM((_N_FIELDS, _CH, _DIM), jnp.float32),
            pltpu.SemaphoreType.DMA,
            pltpu.SemaphoreType.DMA,
        ],
    )
    def sc_combine(idx_hbm, dense_hbm, tables_hbm, out_hbm,
                   idx_v, tmp_v, gsem, wsem):
        wid = lax.axis_index("s") * NC + lax.axis_index("c")
        base = wid * rows_per_w

        dense_copies = [
            pltpu.async_copy(
                dense_hbm.at[pl.ds(base, rows_per_w), :],
                out_hbm.at[pl.ds(base, rows_per_w), pl.ds(_EMB_W, 16)],
                wsem),
        ]

        def chunk_body(c, carry):
            rowbase = base + c * _CH
            pltpu.sync_copy(idx_hbm.at[:, pl.ds(rowbase, _CH)], idx_v)
            gathers = [
                pltpu.async_copy(tables_hbm.at[f].at[idx_v.at[f]],
                                 tmp_v.at[f], gsem)
                for f in range(_N_FIELDS)
            ]
            writes = []
            for f in range(_N_FIELDS):
                gathers[f].wait()
                writes.append(pltpu.async_copy(
                    tmp_v.at[f],
                    out_hbm.at[pl.ds(rowbase, _CH), pl.ds(f * _DIM, _DIM)],
                    wsem))
            for w in writes:
                w.wait()
            return carry

        lax.fori_loop(0, n_chunks, chunk_body, None)
        for cp in dense_copies:
            cp.wait()

    tables_lin = _relayout_tables(tables)
    dense_t = jnp.pad(jnp.transpose(dense), ((0, 0), (0, 3)))
    return sc_combine(indices, dense_t, tables_lin)[:, :_OUT_W]


# speedup vs baseline: 1.8298x; 1.8298x over previous
import functools

import jax
import jax.numpy as jnp
from jax import lax
from jax.experimental import pallas as pl
from jax.experimental.pallas import tpu as pltpu
from jax.experimental.pallas import tpu_sc as plsc

_N_FIELDS = 26
_N_DENSE = 13
_DIM = 32
_EMB_W = _N_FIELDS * _DIM
_OUT_W = _EMB_W + _N_DENSE
_PAD_W = _EMB_W + 16
_CH = 128


def kernel(indices, dense, tables):
    B = indices.shape[1]
    info = plsc.get_sparse_core_info()
    NC, NS = info.num_cores, info.num_subcores
    NW = NC * NS
    rows_per_w = B // NW
    n_chunks = rows_per_w // _CH

    mesh = plsc.VectorSubcoreMesh(core_axis_name="c", subcore_axis_name="s")

    @functools.partial(
        pl.kernel,
        mesh=mesh,
        compiler_params=pltpu.CompilerParams(use_tc_tiling_on_sc=False),
        out_type=jax.ShapeDtypeStruct((B, _PAD_W), jnp.float32),
        scratch_types=[
            pltpu.VMEM((_N_FIELDS, _CH), jnp.int32),
            pltpu.VMEM((_N_FIELDS, _CH, _DIM), jnp.float32),
            pltpu.SemaphoreType.DMA,
            pltpu.SemaphoreType.DMA,
        ],
    )
    def sc_combine(idx_hbm, dense_hbm, tables_hbm, out_hbm,
                   idx_v, tmp_v, gsem, wsem):
        wid = lax.axis_index("s") * NC + lax.axis_index("c")
        base = wid * rows_per_w

        dense_copies = [
            pltpu.async_copy(
                dense_hbm.at[pl.ds(base, rows_per_w), :],
                out_hbm.at[pl.ds(base, rows_per_w), pl.ds(_EMB_W, 16)],
                wsem),
        ]

        def chunk_body(c, carry):
            rowbase = base + c * _CH
            pltpu.sync_copy(idx_hbm.at[:, pl.ds(rowbase, _CH)], idx_v)
            gathers = [
                pltpu.async_copy(tables_hbm.at[f].at[idx_v.at[f]],
                                 tmp_v.at[f], gsem)
                for f in range(_N_FIELDS)
            ]
            writes = []
            for f in range(_N_FIELDS):
                gathers[f].wait()
                writes.append(pltpu.async_copy(
                    tmp_v.at[f],
                    out_hbm.at[pl.ds(rowbase, _CH), pl.ds(f * _DIM, _DIM)],
                    wsem))
            for w in writes:
                w.wait()
            return carry

        lax.fori_loop(0, n_chunks, chunk_body, None)
        for cp in dense_copies:
            cp.wait()

    dense_t = jnp.pad(jnp.transpose(dense), ((0, 0), (0, 3)))
    return sc_combine(indices, dense_t, tables)[:, :_OUT_W]
